# baseline (speedup 1.0000x reference)
import jax
import jax.numpy as jnp
from jax import lax
from jax.experimental import pallas as pl
from jax.experimental.pallas import tpu as pltpu

V = 100000
E = 128
H = 128
CTX = 200

BV = 16384
NB = -(-V // BV)
VP = NB * BV


def _tc_body(idx_r, W1r, b1r, W2r, b2r, table_r, outr,
             logits_s, rows_s, h_s, m_s, s_s, gsem):
    i = pl.program_id(0)

    @pl.when(i == 0)
    def _():

        def _start(j, _):
            row = idx_r[j]
            pltpu.make_async_copy(table_r.at[pl.ds(row, 1)],
                                  rows_s.at[pl.ds(j, 1)], gsem).start()
            return 0
        lax.fori_loop(0, CTX, _start, 0, unroll=8)

        pltpu.make_async_copy(table_r.at[pl.ds(0, CTX)], rows_s, gsem).wait()

        emb = jnp.sum(rows_s[...], axis=0, keepdims=True)
        hh = lax.dot_general(emb, W1r[...], (((1,), (1,)), ((), ())),
                             preferred_element_type=jnp.float32) + b1r[...]
        hh = jnp.maximum(hh, 0.0)
        hc = hh.astype(jnp.bfloat16)
        hr = (hh - hc.astype(jnp.float32)).astype(jnp.bfloat16)
        h_s[...] = jnp.concatenate([hc, hr], axis=0)
        m_s[...] = jnp.full((1, 1), -1e30, jnp.float32)
        s_s[...] = jnp.zeros((1, 1), jnp.float32)

    lg2 = lax.dot_general(h_s[...], W2r[...].astype(jnp.bfloat16),
                          (((1,), (1,)), ((), ())),
                          preferred_element_type=jnp.float32)
    logits = lg2[0:1, :] + lg2[1:2, :] + b2r[...].reshape(1, BV)
    logits_s[:, pl.ds(pl.multiple_of(i * BV, 128), BV)] = logits

    col = i * BV + lax.broadcasted_iota(jnp.int32, (1, BV), 1)
    lm = jnp.where(col < V, logits, -1e30)

    m_old = m_s[...]
    bm = jnp.max(lm, axis=(0, 1), keepdims=True)
    m_new = jnp.maximum(m_old, bm)
    s_s[...] = (s_s[...] * jnp.exp(m_old - m_new)
                + jnp.sum(jnp.exp(lm - m_new), axis=(0, 1), keepdims=True))
    m_s[...] = m_new

    @pl.when(i == NB - 1)
    def _():
        lse = m_new + jnp.log(s_s[...])
        outr[...] = logits_s[:, :V] - lse


_tc_main = pl.pallas_call(
    _tc_body,
    grid=(NB,),
    in_specs=[
        pl.BlockSpec(memory_space=pltpu.SMEM),
        pl.BlockSpec((H, E), lambda i: (0, 0)),
        pl.BlockSpec((1, H), lambda i: (0, 0)),
        pl.BlockSpec((BV, H), lambda i: (i, 0)),
        pl.BlockSpec((BV,), lambda i: (i,)),
        pl.BlockSpec(memory_space=pl.ANY),
    ],
    out_specs=pl.BlockSpec((1, V), lambda i: (0, 0)),
    out_shape=jax.ShapeDtypeStruct((1, V), jnp.float32),
    scratch_shapes=[
        pltpu.VMEM((1, VP), jnp.float32),
        pltpu.VMEM((CTX, E), jnp.float32),
        pltpu.VMEM((2, H), jnp.bfloat16),
        pltpu.VMEM((1, 1), jnp.float32),
        pltpu.VMEM((1, 1), jnp.float32),
        pltpu.SemaphoreType.DMA,
    ],
)


def kernel(inputs, table, W1, b1, W2, b2):
    idx = inputs.astype(jnp.int32)
    return _tc_main(idx, W1, b1.reshape(1, H), W2, b2, table)


# speedup vs baseline: 1.0150x; 1.0150x over previous
import jax
import jax.numpy as jnp
from jax import lax
from jax.experimental import pallas as pl
from jax.experimental.pallas import tpu as pltpu

V = 100000
E = 128
H = 128
CTX = 200

BV = 18432
NB = -(-V // BV)
VP = NB * BV


def _tc_body(idx_r, W1r, b1r, W2r, b2r, table_r, outr,
             logits_s, rows_s, h_s, m_s, s_s, gsem):
    i = pl.program_id(0)

    @pl.when(i == 0)
    def _():

        def _start(j, _):
            row = idx_r[j]
            pltpu.make_async_copy(table_r.at[pl.ds(row, 1)],
                                  rows_s.at[pl.ds(j, 1)], gsem).start()
            return 0
        lax.fori_loop(0, CTX, _start, 0, unroll=8)

        pltpu.make_async_copy(table_r.at[pl.ds(0, CTX)], rows_s, gsem).wait()

        emb = jnp.sum(rows_s[...], axis=0, keepdims=True)
        hh = lax.dot_general(emb, W1r[...], (((1,), (1,)), ((), ())),
                             preferred_element_type=jnp.float32) + b1r[...]
        hh = jnp.maximum(hh, 0.0)
        hc = hh.astype(jnp.bfloat16)
        hr = (hh - hc.astype(jnp.float32)).astype(jnp.bfloat16)
        h_s[...] = jnp.concatenate([hc, hr], axis=0)
        m_s[...] = jnp.full((1, 1), -1e30, jnp.float32)
        s_s[...] = jnp.zeros((1, 1), jnp.float32)

    lg2 = lax.dot_general(h_s[...], W2r[...].astype(jnp.bfloat16),
                          (((1,), (1,)), ((), ())),
                          preferred_element_type=jnp.float32)
    logits = lg2[0:1, :] + lg2[1:2, :] + b2r[...].reshape(1, BV)
    logits_s[:, pl.ds(pl.multiple_of(i * BV, 128), BV)] = logits

    col = i * BV + lax.broadcasted_iota(jnp.int32, (1, BV), 1)
    lm = jnp.where(col < V, logits, -1e30)

    m_old = m_s[...]
    bm = jnp.max(lm, axis=(0, 1), keepdims=True)
    m_new = jnp.maximum(m_old, bm)
    s_s[...] = (s_s[...] * jnp.exp(m_old - m_new)
                + jnp.sum(jnp.exp(lm - m_new), axis=(0, 1), keepdims=True))
    m_s[...] = m_new

    @pl.when(i == NB - 1)
    def _():
        lse = m_new + jnp.log(s_s[...])
        outr[...] = logits_s[:, :V] - lse


_tc_main = pl.pallas_call(
    _tc_body,
    grid=(NB,),
    in_specs=[
        pl.BlockSpec(memory_space=pltpu.SMEM),
        pl.BlockSpec((H, E), lambda i: (0, 0)),
        pl.BlockSpec((1, H), lambda i: (0, 0)),
        pl.BlockSpec((BV, H), lambda i: (i, 0)),
        pl.BlockSpec((BV,), lambda i: (i,)),
        pl.BlockSpec(memory_space=pl.ANY),
    ],
    out_specs=pl.BlockSpec((1, V), lambda i: (0, 0)),
    out_shape=jax.ShapeDtypeStruct((1, V), jnp.float32),
    scratch_shapes=[
        pltpu.VMEM((1, VP), jnp.float32),
        pltpu.VMEM((CTX, E), jnp.float32),
        pltpu.VMEM((2, H), jnp.bfloat16),
        pltpu.VMEM((1, 1), jnp.float32),
        pltpu.VMEM((1, 1), jnp.float32),
        pltpu.SemaphoreType.DMA,
    ],
)


def kernel(inputs, table, W1, b1, W2, b2):
    idx = inputs.astype(jnp.int32)
    return _tc_main(idx, W1, b1.reshape(1, H), W2, b2, table)


# speedup vs baseline: 1.0545x; 1.0390x over previous
import jax
import jax.numpy as jnp
from jax import lax
from jax.experimental import pallas as pl
from jax.experimental.pallas import tpu as pltpu

V = 100000
E = 128
H = 128
CTX = 200

BV = 20480
NB = -(-V // BV)
VP = NB * BV


def _tc_body(idx_r, W1r, b1r, W2r, b2r, table_r, outr,
             logits_s, rows_s, h_s, m_s, s_s, gsem):
    i = pl.program_id(0)

    @pl.when(i == 0)
    def _():

        def _start(j, _):
            row = idx_r[j]
            pltpu.make_async_copy(table_r.at[pl.ds(row, 1)],
                                  rows_s.at[pl.ds(j, 1)], gsem).start()
            return 0
        lax.fori_loop(0, CTX, _start, 0, unroll=8)

        pltpu.make_async_copy(table_r.at[pl.ds(0, CTX)], rows_s, gsem).wait()

        emb = jnp.sum(rows_s[...], axis=0, keepdims=True)
        hh = lax.dot_general(emb, W1r[...], (((1,), (1,)), ((), ())),
                             preferred_element_type=jnp.float32) + b1r[...]
        hh = jnp.maximum(hh, 0.0)
        hc = hh.astype(jnp.bfloat16)
        hr = (hh - hc.astype(jnp.float32)).astype(jnp.bfloat16)
        h_s[...] = jnp.concatenate([hc, hr], axis=0)
        m_s[...] = jnp.full((1, 1), -1e30, jnp.float32)
        s_s[...] = jnp.zeros((1, 1), jnp.float32)

    lg2 = lax.dot_general(h_s[...], W2r[...].astype(jnp.bfloat16),
                          (((1,), (1,)), ((), ())),
                          preferred_element_type=jnp.float32)
    logits = lg2[0:1, :] + lg2[1:2, :] + b2r[...].reshape(1, BV)
    logits_s[:, pl.ds(pl.multiple_of(i * BV, 128), BV)] = logits

    col = i * BV + lax.broadcasted_iota(jnp.int32, (1, BV), 1)
    lm = jnp.where(col < V, logits, -1e30)

    m_old = m_s[...]
    bm = jnp.max(lm, axis=(0, 1), keepdims=True)
    m_new = jnp.maximum(m_old, bm)
    s_s[...] = (s_s[...] * jnp.exp(m_old - m_new)
                + jnp.sum(jnp.exp(lm - m_new), axis=(0, 1), keepdims=True))
    m_s[...] = m_new

    @pl.when(i == NB - 1)
    def _():
        lse = m_new + jnp.log(s_s[...])
        outr[...] = logits_s[:, :V] - lse


_tc_main = pl.pallas_call(
    _tc_body,
    grid=(NB,),
    in_specs=[
        pl.BlockSpec(memory_space=pltpu.SMEM),
        pl.BlockSpec((H, E), lambda i: (0, 0)),
        pl.BlockSpec((1, H), lambda i: (0, 0)),
        pl.BlockSpec((BV, H), lambda i: (i, 0)),
        pl.BlockSpec((BV,), lambda i: (i,)),
        pl.BlockSpec(memory_space=pl.ANY),
    ],
    out_specs=pl.BlockSpec((1, V), lambda i: (0, 0)),
    out_shape=jax.ShapeDtypeStruct((1, V), jnp.float32),
    scratch_shapes=[
        pltpu.VMEM((1, VP), jnp.float32),
        pltpu.VMEM((CTX, E), jnp.float32),
        pltpu.VMEM((2, H), jnp.bfloat16),
        pltpu.VMEM((1, 1), jnp.float32),
        pltpu.VMEM((1, 1), jnp.float32),
        pltpu.SemaphoreType.DMA,
    ],
)


def kernel(inputs, table, W1, b1, W2, b2):
    idx = inputs.astype(jnp.int32)
    return _tc_main(idx, W1, b1.reshape(1, H), W2, b2, table)
